# baseline (speedup 1.0000x reference)
import functools

import jax
import jax.numpy as jnp
from jax import lax
from jax.experimental import pallas as pl
from jax.experimental.pallas import tpu as pltpu
from jax.experimental.pallas import tpu_sc as plsc

BATCH = 4
SEQ_LEN = 4096
EMBED_DIM = 1024

NUM_CORES = 2
NUM_SUBCORES = 16
NUM_WORKERS = NUM_CORES * NUM_SUBCORES
ROW_BLOCK = 32
BLOCK_ELEMS = ROW_BLOCK * EMBED_DIM
LANES = 16


def _sc_add(x_hbm, pos_hbm, out_hbm, posbuf, xbuf):
    seq_per_worker = SEQ_LEN // NUM_WORKERS
    wid = lax.axis_index("c") * NUM_SUBCORES + lax.axis_index("s")
    s0 = wid * seq_per_worker
    for j in range(seq_per_worker // ROW_BLOCK):
        sj = s0 + j * ROW_BLOCK
        pltpu.sync_copy(pos_hbm.at[pl.ds(sj * EMBED_DIM, BLOCK_ELEMS)], posbuf)
        for b in range(BATCH):
            off = (b * SEQ_LEN + sj) * EMBED_DIM
            pltpu.sync_copy(x_hbm.at[pl.ds(off, BLOCK_ELEMS)], xbuf)

            @plsc.parallel_loop(0, BLOCK_ELEMS, LANES, unroll=8)
            def _(i):
                plsc.addupdate(xbuf.at[pl.ds(i, LANES)], posbuf[pl.ds(i, LANES)])

            pltpu.sync_copy(xbuf, out_hbm.at[pl.ds(off, BLOCK_ELEMS)])


def kernel(x, pos_table):
    batch, seq_len, embed_dim = x.shape
    x1 = x.reshape(-1)
    pos1 = pos_table[:seq_len].reshape(-1)
    k = functools.partial(
        pl.kernel,
        out_type=jax.ShapeDtypeStruct((batch * seq_len * embed_dim,), x.dtype),
        mesh=plsc.VectorSubcoreMesh(core_axis_name="c", subcore_axis_name="s"),
        scratch_types=[
            pltpu.VMEM((BLOCK_ELEMS,), jnp.float32),
            pltpu.VMEM((BLOCK_ELEMS,), jnp.float32),
        ],
    )(_sc_add)
    return k(x1, pos1).reshape(batch, seq_len, embed_dim)


# speedup vs baseline: 5.1330x; 5.1330x over previous
import jax
import jax.numpy as jnp
from jax.experimental import pallas as pl

BATCH = 4
SEQ_LEN = 4096
EMBED_DIM = 1024
SEQ_BLOCK = 1024


def _add_block(x_ref, pos_ref, o_ref):
    o_ref[...] = x_ref[...] + pos_ref[...]


def kernel(x, pos_table):
    batch, seq_len, embed_dim = x.shape
    n_seq = seq_len // SEQ_BLOCK
    pos = pos_table[:seq_len]
    return pl.pallas_call(
        _add_block,
        grid=(n_seq, batch),
        in_specs=[
            pl.BlockSpec((1, SEQ_BLOCK, embed_dim), lambda i, j: (j, i, 0)),
            pl.BlockSpec((SEQ_BLOCK, embed_dim), lambda i, j: (i, 0)),
        ],
        out_specs=pl.BlockSpec((1, SEQ_BLOCK, embed_dim), lambda i, j: (j, i, 0)),
        out_shape=jax.ShapeDtypeStruct((batch, seq_len, embed_dim), x.dtype),
    )(x, pos)
